# Initial kernel scaffold; baseline (speedup 1.0000x reference)
import jax
import jax.numpy as jnp
from jax import lax
from jax.experimental import pallas as pl
from jax.experimental.pallas import tpu as pltpu
from jax.experimental.pallas import tpu_sc as plsc

N = 10000
E = 320000
D = 128
NC = 2
NS = 16
NW = NC * NS
EPW = E // NW
C = 80
NCH = EPW // C
RPT = N // NS
ZROWS = 125
SEG = D // 16


def _sc_body(x_hbm, src_hbm, tgt_hbm, ew_hbm, out_hbm,
             src_v, tgt_v, ew_v, rows_v, zbuf_v, delta_sh, gsem):
    cid = lax.axis_index("c")
    sid = lax.axis_index("s")
    wid = cid * NS + sid

    pltpu.sync_copy(src_hbm.at[wid], src_v)
    pltpu.sync_copy(tgt_hbm.at[wid], tgt_v)
    pltpu.sync_copy(ew_hbm.at[wid], ew_v)

    zeros16 = jnp.zeros((16,), jnp.float32)

    @plsc.parallel_loop(0, ZROWS)
    def _zero_row(r):
        for s in range(SEG):
            zbuf_v[r, pl.ds(s * 16, 16)] = zeros16

    row0 = sid * RPT
    for k in range(RPT // ZROWS):
        pltpu.sync_copy(zbuf_v, delta_sh.at[pl.ds(row0 + k * ZROWS, ZROWS)])
    plsc.subcore_barrier()

    def chunk_body(j, carry):
        pltpu.async_copy(x_hbm.at[src_v.at[j]], rows_v, gsem).wait()

        @plsc.parallel_loop(0, C, unroll=2)
        def _scale(e):
            w = ew_v[j, e]
            for s in range(SEG):
                sl = pl.ds(s * 16, 16)
                rows_v[e, sl] = rows_v[e, sl] * w

        pltpu.sync_copy(rows_v, delta_sh.at[tgt_v.at[j]], add=True)
        return carry

    lax.fori_loop(0, NCH, chunk_body, 0)
    plsc.subcore_barrier()

    for k in range(RPT // ZROWS):
        r = row0 + k * ZROWS
        pltpu.sync_copy(delta_sh.at[pl.ds(r, ZROWS)], zbuf_v)
        pltpu.sync_copy(zbuf_v, out_hbm.at[cid].at[pl.ds(r, ZROWS)])


_sc_scatter = pl.kernel(
    _sc_body,
    out_type=jax.ShapeDtypeStruct((NC, N, D), jnp.float32),
    mesh=plsc.VectorSubcoreMesh(core_axis_name="c", subcore_axis_name="s"),
    scratch_types=[
        pltpu.VMEM((NCH, C), jnp.int32),
        pltpu.VMEM((NCH, C), jnp.int32),
        pltpu.VMEM((NCH, C), jnp.float32),
        pltpu.VMEM((C, D), jnp.float32),
        pltpu.VMEM((ZROWS, D), jnp.float32),
        pltpu.VMEM_SHARED((N, D), jnp.float32),
        pltpu.SemaphoreType.DMA,
    ],
)


BLK = 400


def _tc_body(x_ref, d0_ref, d1_ref, ws_ref, wd_ref, b_ref, o_ref):
    d = d0_ref[...] + d1_ref[...]
    o_ref[...] = (
        jnp.dot(x_ref[...], ws_ref[...], preferred_element_type=jnp.float32)
        + jnp.dot(d, wd_ref[...], preferred_element_type=jnp.float32)
        + b_ref[...]
    )


_node_update = pl.pallas_call(
    _tc_body,
    grid=(N // BLK,),
    in_specs=[
        pl.BlockSpec((BLK, D), lambda i: (i, 0)),
        pl.BlockSpec((BLK, D), lambda i: (i, 0)),
        pl.BlockSpec((BLK, D), lambda i: (i, 0)),
        pl.BlockSpec((D, D), lambda i: (0, 0)),
        pl.BlockSpec((D, D), lambda i: (0, 0)),
        pl.BlockSpec((1, D), lambda i: (0, 0)),
    ],
    out_specs=pl.BlockSpec((BLK, D), lambda i: (i, 0)),
    out_shape=jax.ShapeDtypeStruct((N, D), jnp.float32),
)


@jax.jit
def kernel(x, source, target, edge_weights, W_self, W_delta, b):
    src3 = source.reshape(NW, NCH, C)
    tgt3 = target.reshape(NW, NCH, C)
    ew3 = edge_weights.reshape(NW, NCH, C)
    partial = _sc_scatter(x, src3, tgt3, ew3)
    return _node_update(x, partial[0], partial[1], W_self, W_delta,
                        b.reshape(1, D))


# speedup vs baseline: 3.6707x; 3.6707x over previous
import jax
import jax.numpy as jnp
from jax import lax
from jax.experimental import pallas as pl
from jax.experimental.pallas import tpu as pltpu
from jax.experimental.pallas import tpu_sc as plsc

N = 10000
E = 320000
D = 128
NC = 2
NS = 16
NW = NC * NS
EPW = E // NW
C = 80
NCH = EPW // C
PR = 5120
NP = 5632
RPT = NP // NS
ZCH = 88
DPT = PR // NS
DCH = 80
SEG = D // 16


def _sc_body(x_hbm, src_hbm, tgt_hbm, ew_hbm, out_hbm,
             src_v, tgt_v, ew_v, rows_v, tloc_v, zbuf_v, delta_sh, gsem):
    cid = lax.axis_index("c")
    sid = lax.axis_index("s")
    wid = cid * NS + sid

    pltpu.sync_copy(src_hbm.at[wid], src_v)
    pltpu.sync_copy(tgt_hbm.at[wid], tgt_v)
    pltpu.sync_copy(ew_hbm.at[wid], ew_v)

    zeros16 = jnp.zeros((16,), jnp.float32)

    for p in range(2):
        lo = p * PR

        @plsc.parallel_loop(0, ZCH)
        def _zero_row(r):
            for s in range(SEG):
                zbuf_v[r, pl.ds(s * 16, 16)] = zeros16

        for k in range(RPT // ZCH):
            pltpu.sync_copy(zbuf_v, delta_sh.at[pl.ds(sid * RPT + k * ZCH, ZCH)])
        plsc.subcore_barrier()

        def chunk_body(j, carry):
            pltpu.async_copy(x_hbm.at[src_v.at[j]], rows_v, gsem).wait()

            @plsc.parallel_loop(0, C, step=16)
            def _scale(g):
                t16 = tgt_v[j, pl.ds(g, 16)] - lo
                ok = (t16 >= 0) & (t16 < PR)
                tloc_v[0, pl.ds(g, 16)] = jnp.where(ok, t16, PR)
                w16 = ew_v[j, pl.ds(g, 16)]
                for l in range(16):
                    w = w16[l]
                    for s in range(SEG):
                        sl = pl.ds(s * 16, 16)
                        rows_v[g + l, sl] = rows_v[g + l, sl] * w

            pltpu.sync_copy(rows_v, delta_sh.at[tloc_v.at[0]], add=True)
            return carry

        lax.fori_loop(0, NCH, chunk_body, 0)
        plsc.subcore_barrier()

        for k in range(DPT // DCH):
            r = sid * DPT + k * DCH

            @pl.when(lo + r < N)
            def _drain():
                pltpu.sync_copy(delta_sh.at[pl.ds(r, DCH)],
                                zbuf_v.at[pl.ds(0, DCH)])
                pltpu.sync_copy(zbuf_v.at[pl.ds(0, DCH)],
                                out_hbm.at[cid].at[pl.ds(lo + r, DCH)])

        if p == 0:
            plsc.subcore_barrier()


_sc_scatter = pl.kernel(
    _sc_body,
    out_type=jax.ShapeDtypeStruct((NC, N, D), jnp.float32),
    mesh=plsc.VectorSubcoreMesh(core_axis_name="c", subcore_axis_name="s"),
    scratch_types=[
        pltpu.VMEM((NCH, C), jnp.int32),
        pltpu.VMEM((NCH, C), jnp.int32),
        pltpu.VMEM((NCH, C), jnp.float32),
        pltpu.VMEM((C, D), jnp.float32),
        pltpu.VMEM((1, C), jnp.int32),
        pltpu.VMEM((ZCH, D), jnp.float32),
        pltpu.VMEM_SHARED((NP, D), jnp.float32),
        pltpu.SemaphoreType.DMA,
    ],
)


BLK = 400


def _tc_body(x_ref, d0_ref, d1_ref, ws_ref, wd_ref, b_ref, o_ref):
    d = d0_ref[...] + d1_ref[...]
    o_ref[...] = (
        jnp.dot(x_ref[...], ws_ref[...], preferred_element_type=jnp.float32)
        + jnp.dot(d, wd_ref[...], preferred_element_type=jnp.float32)
        + b_ref[...]
    )


_node_update = pl.pallas_call(
    _tc_body,
    grid=(N // BLK,),
    in_specs=[
        pl.BlockSpec((BLK, D), lambda i: (i, 0)),
        pl.BlockSpec((BLK, D), lambda i: (i, 0)),
        pl.BlockSpec((BLK, D), lambda i: (i, 0)),
        pl.BlockSpec((D, D), lambda i: (0, 0)),
        pl.BlockSpec((D, D), lambda i: (0, 0)),
        pl.BlockSpec((1, D), lambda i: (0, 0)),
    ],
    out_specs=pl.BlockSpec((BLK, D), lambda i: (i, 0)),
    out_shape=jax.ShapeDtypeStruct((N, D), jnp.float32),
)


@jax.jit
def kernel(x, source, target, edge_weights, W_self, W_delta, b):
    src3 = source.reshape(NW, NCH, C)
    tgt3 = target.reshape(NW, NCH, C)
    ew3 = edge_weights.reshape(NW, NCH, C)
    d = _sc_scatter(x, src3, tgt3, ew3)
    return _node_update(x, d[0], d[1], W_self, W_delta, b.reshape(1, D))
